# baseline (speedup 1.0000x reference)
import functools
import jax
import jax.numpy as jnp
from jax import lax
from jax.experimental import pallas as pl
from jax.experimental.pallas import tpu as pltpu
from jax.experimental.pallas import tpu_sc as plsc

L = 20
B = 16384
N_STATES = 2 ** L

_NC = 2
_NS = 16
_NW = _NC * _NS
_BPW = B // _NW
_NCHUNK = _BPW // 128


def _sc_body(x_hbm, table_hbm, out_hbm, xv, idxv, outv, sem):
    wid = lax.axis_index("s") * _NC + lax.axis_index("c")
    base = wid * _BPW

    with jax.named_scope("stage_x"):
        pltpu.sync_copy(x_hbm.at[pl.ds(base * L, _BPW * L)], xv)

    lane = lax.iota(jnp.int32, 16)

    bias = jnp.full((16,), 2.0 ** (L - 1) - 0.5, jnp.float32)

    copies = []
    for c in range(_NCHUNK):
        def body(j, _):
            flat0 = (lane + (c * 8 + j) * 16) * L
            accs = [None, None, None, None]
            for l in range(L):
                v = plsc.load_gather(xv, [flat0 + l])
                term = v * jnp.float32(2.0 ** (L - 2 - l))
                a = l % 4
                accs[a] = term if accs[a] is None else accs[a] + term
            acc = (accs[0] + accs[1]) + (accs[2] + accs[3]) + bias
            idxv[c, pl.ds(j * 16, 16)] = acc.astype(jnp.int32)
            return 0

        with jax.named_scope(f"compute{c}"):
            lax.fori_loop(0, 8, body, 0, unroll=2)
        copies.append(
            pltpu.async_copy(
                table_hbm.at[idxv.at[c]], outv.at[pl.ds(c * 128, 128)], sem
            )
        )

    with jax.named_scope("gather_wait"):
        for cp in copies:
            cp.wait()
    with jax.named_scope("write_out"):
        pltpu.sync_copy(outv, out_hbm.at[pl.ds(base, _BPW)])


@functools.partial(jax.jit, static_argnames=())
def kernel(x_in, logstate):
    mesh = plsc.VectorSubcoreMesh(core_axis_name="c", subcore_axis_name="s")
    run = pl.kernel(
        _sc_body,
        mesh=mesh,
        out_type=jax.ShapeDtypeStruct((B,), jnp.float32),
        scratch_types=[
            pltpu.VMEM((_BPW * L,), jnp.float32),
            pltpu.VMEM((_NCHUNK, 128), jnp.int32),
            pltpu.VMEM((_BPW,), jnp.float32),
            pltpu.SemaphoreType.DMA,
        ],
        compiler_params=pltpu.CompilerParams(needs_layout_passes=False),
    )
    return run(x_in.reshape(-1), logstate)


# speedup vs baseline: 1.1279x; 1.1279x over previous
import functools
import jax
import jax.numpy as jnp
from jax import lax
from jax.experimental import pallas as pl
from jax.experimental.pallas import tpu as pltpu
from jax.experimental.pallas import tpu_sc as plsc

L = 20
B = 16384
N_STATES = 2 ** L

_NC = 2
_NS = 16
_NW = _NC * _NS
_BPW = B // _NW
_NCHUNK = _BPW // 128


def _sc_body(x_hbm, table_hbm, out_hbm, xv, idxv, outv, sem):
    wid = lax.axis_index("s") * _NC + lax.axis_index("c")
    base = wid * _BPW

    pltpu.sync_copy(x_hbm.at[pl.ds(base, _BPW), :], xv)

    lane = lax.iota(jnp.int32, 16)

    bias = jnp.full((16,), 2.0 ** (L - 1) - 0.5, jnp.float32)

    def body(j, _):
        row = lane + j * 16
        accs = [None, None, None, None]
        for l in range(L):
            v = plsc.load_gather(xv, [row, jnp.full((16,), l, jnp.int32)])
            term = v * jnp.float32(2.0 ** (L - 2 - l))
            a = l % 4
            accs[a] = term if accs[a] is None else accs[a] + term
        acc = (accs[0] + accs[1]) + (accs[2] + accs[3]) + bias
        idxv[j // 8, pl.ds((j % 8) * 16, 16)] = acc.astype(jnp.int32)
        return 0

    lax.fori_loop(0, 8 * _NCHUNK, body, 0)

    copies = [
        pltpu.async_copy(
            table_hbm.at[idxv.at[c]], outv.at[pl.ds(c * 128, 128)], sem
        )
        for c in range(_NCHUNK)
    ]
    for cp in copies:
        cp.wait()
    pltpu.sync_copy(outv, out_hbm.at[pl.ds(base, _BPW)])


@jax.jit
def kernel(x_in, logstate):
    mesh = plsc.VectorSubcoreMesh(core_axis_name="c", subcore_axis_name="s")
    run = pl.kernel(
        _sc_body,
        mesh=mesh,
        out_type=jax.ShapeDtypeStruct((B,), jnp.float32),
        scratch_types=[
            pltpu.VMEM((_BPW, L), jnp.float32),
            pltpu.VMEM((_NCHUNK, 128), jnp.int32),
            pltpu.VMEM((_BPW,), jnp.float32),
            pltpu.SemaphoreType.DMA,
        ],
        compiler_params=pltpu.CompilerParams(needs_layout_passes=False),
    )
    return run(x_in, logstate)
